# baseline (speedup 1.0000x reference)
import jax
import jax.numpy as jnp
from jax import lax
from jax.experimental import pallas as pl
from jax.experimental.pallas import tpu as pltpu
from jax.experimental.pallas import tpu_sc as plsc

NC = 2
NS = 16
NW = NC * NS
CHUNK = 32
NSLOT = 4


def _make_agg(chunks, n_pad, with_deg):
  rps = n_pad // NS
  assert rps % CHUNK == 0
  mesh = plsc.VectorSubcoreMesh(core_axis_name="c", subcore_axis_name="s")

  out_type = [jax.ShapeDtypeStruct((NC, n_pad, 128), jnp.float32)]
  scratch = [
      pltpu.VMEM((chunks, CHUNK), jnp.int32),
      pltpu.VMEM((chunks, CHUNK), jnp.int32),
      pltpu.VMEM((NSLOT * CHUNK, 128), jnp.float32),
      pltpu.VMEM_SHARED((n_pad, 128), jnp.float32),
  ] + [pltpu.SemaphoreType.DMA] * (2 * NSLOT)
  if with_deg:
    out_type.append(jax.ShapeDtypeStruct((NW, n_pad), jnp.float32))
    scratch.append(pltpu.VMEM((n_pad,), jnp.float32))

  def body(*refs):
    if with_deg:
      (feat, srcp, dstp, p_out, deg_out,
       src_v, dst_v, buf0, acc_sh, *rest) = refs
    else:
      (feat, srcp, dstp, p_out,
       src_v, dst_v, buf0, acc_sh, *rest) = refs
    gs = rest[:NSLOT]
    ss = rest[NSLOT:2 * NSLOT]
    if with_deg:
      deg_v = rest[2 * NSLOT]
    slots = [buf0.at[pl.ds(b * CHUNK, CHUNK)] for b in range(NSLOT)]

    c = lax.axis_index("c")
    s = lax.axis_index("s")
    wid = c * NS + s
    r0 = s * rps

    pltpu.sync_copy(srcp.at[wid], src_v)
    pltpu.sync_copy(dstp.at[wid], dst_v)

    z16 = jnp.zeros((16,), jnp.float32)

    bufrows = NSLOT * CHUNK

    def zrow(i, carry):
      for k in range(8):
        buf0[i, pl.ds(k * 16, 16)] = z16
      return carry

    lax.fori_loop(0, bufrows, zrow, 0)
    for i in range(rps // bufrows):
      pltpu.sync_copy(buf0, acc_sh.at[pl.ds(r0 + i * bufrows, bufrows)])

    if with_deg:
      def dzrow(i, carry):
        deg_v[pl.ds(i * 16, 16)] = z16
        return carry

      lax.fori_loop(0, n_pad // 16, dzrow, 0)

    plsc.subcore_barrier()

    one16 = jnp.ones((16,), jnp.float32)

    def gstart(j, b):
      pltpu.async_copy(feat.at[src_v.at[j]], slots[b], gs[b])

    def gwait(j, b):
      pltpu.make_async_copy(feat.at[src_v.at[j]], slots[b], gs[b]).wait()

    def sstart(j, b):
      pltpu.async_copy(slots[b], acc_sh.at[dst_v.at[j]], ss[b], add=True)

    def swait(j, b):
      pltpu.make_async_copy(slots[b], acc_sh.at[dst_v.at[j]], ss[b]).wait()

    def deg_hist(j):
      if with_deg:
        for k in range(CHUNK // 16):
          idx = dst_v[j, pl.ds(k * 16, 16)]
          plsc.addupdate_scatter(deg_v, [idx], one16)

    gstart(0, 0)
    gstart(1, 1)

    def step(i, carry):
      j0 = NSLOT * i
      for b in range(NSLOT):
        j = j0 + b
        gwait(j, b)
        sstart(j, b)
        deg_hist(j)
        b2 = (b + 2) % NSLOT
        jprev = j - 2

        @pl.when(jprev >= 0)
        def _():
          swait(jprev, b2)

        @pl.when(j + 2 < chunks)
        def _():
          gstart(j + 2, b2)

      return carry

    lax.fori_loop(0, chunks // NSLOT, step, 0)
    swait(chunks - 2, (chunks - 2) % NSLOT)
    swait(chunks - 1, (chunks - 1) % NSLOT)

    plsc.subcore_barrier()

    for i in range(rps // bufrows):
      rr = r0 + i * bufrows
      pltpu.sync_copy(acc_sh.at[pl.ds(rr, bufrows)], buf0)
      pltpu.sync_copy(buf0, p_out.at[c, pl.ds(rr, bufrows)])
    if with_deg:
      pltpu.sync_copy(deg_v, deg_out.at[wid])

  return pl.kernel(body, out_type=tuple(out_type), mesh=mesh,
                   scratch_types=tuple(scratch),
                   compiler_params=pltpu.CompilerParams(
                       use_tc_tiling_on_sc=False,
                       needs_layout_passes=False))


def _mm_body(p_ref, dg_ref, w1_ref, b1_ref, w2_ref, o_ref):
  deg = jnp.maximum(jnp.sum(dg_ref[...], axis=0), 1.0)
  inv = (1.0 / deg)[:, None]
  agg = (p_ref[0] + p_ref[1]) * inv
  h = jnp.dot(agg, w1_ref[...], preferred_element_type=jnp.float32)
  h = jnp.maximum(h + b1_ref[...], 0.0)
  o_ref[...] = jnp.dot(h, w2_ref[...], preferred_element_type=jnp.float32)


def _fin_body(q_ref, dg_ref, b2_ref, o_ref):
  deg = jnp.maximum(jnp.sum(dg_ref[...], axis=0), 1.0)
  inv = (1.0 / deg)[:, None]
  o_ref[...] = (q_ref[0] + q_ref[1]) * inv + b2_ref[...]


def kernel(x, edge_index, W1, b1, W2, b2):
  n = x.shape[0]
  e = edge_index.shape[1]
  n_pad = pl.cdiv(n, NS * CHUNK) * NS * CHUNK
  if n_pad == n:
    n_pad += NS * CHUNK
  chunks = pl.cdiv(pl.cdiv(e, NW * CHUNK), NSLOT) * NSLOT
  e_pad = NW * CHUNK * chunks

  src = edge_index[0]
  dst = edge_index[1]
  pidx = jnp.arange(e_pad - e, dtype=jnp.int32)
  pad_src = pidx % n
  pad_dst = n + pidx % (n_pad - n)
  srcp = jnp.concatenate([src, pad_src]).reshape(NW, chunks, CHUNK)
  dstp = jnp.concatenate([dst, pad_dst]).reshape(NW, chunks, CHUNK)

  agg_deg = _make_agg(chunks, n_pad, with_deg=True)
  agg = _make_agg(chunks, n_pad, with_deg=False)

  p, degw = agg_deg(x, srcp, dstp)

  rb = 1280
  grid = (n_pad // rb,)
  h2 = pl.pallas_call(
      _mm_body,
      grid=grid,
      in_specs=[
          pl.BlockSpec((NC, rb, 128), lambda i: (0, i, 0)),
          pl.BlockSpec((NW, rb), lambda i: (0, i)),
          pl.BlockSpec((128, 128), lambda i: (0, 0)),
          pl.BlockSpec((1, 128), lambda i: (0, 0)),
          pl.BlockSpec((128, 128), lambda i: (0, 0)),
      ],
      out_specs=pl.BlockSpec((rb, 128), lambda i: (i, 0)),
      out_shape=jax.ShapeDtypeStruct((n_pad, 128), jnp.float32),
  )(p, degw, W1, b1.reshape(1, 128), W2)

  (q,) = agg(h2, srcp, dstp)

  out = pl.pallas_call(
      _fin_body,
      grid=grid,
      in_specs=[
          pl.BlockSpec((NC, rb, 128), lambda i: (0, i, 0)),
          pl.BlockSpec((NW, rb), lambda i: (0, i)),
          pl.BlockSpec((1, 128), lambda i: (0, 0)),
      ],
      out_specs=pl.BlockSpec((rb, 128), lambda i: (i, 0)),
      out_shape=jax.ShapeDtypeStruct((n_pad, 128), jnp.float32),
  )(q, degw, b2.reshape(1, 128))

  return out[:n]


# speedup vs baseline: 1.1618x; 1.1618x over previous
import jax
import jax.numpy as jnp
from jax import lax
from jax.experimental import pallas as pl
from jax.experimental.pallas import tpu as pltpu
from jax.experimental.pallas import tpu_sc as plsc

NC = 2
NS = 16
NW = NC * NS
CHUNK = 64
NSLOT = 2


def _make_agg(chunks, n_pad, with_deg):
  rps = n_pad // NS
  assert rps % CHUNK == 0
  mesh = plsc.VectorSubcoreMesh(core_axis_name="c", subcore_axis_name="s")

  out_type = [jax.ShapeDtypeStruct((NC, n_pad, 128), jnp.float32)]
  scratch = [
      pltpu.VMEM((chunks, CHUNK), jnp.int32),
      pltpu.VMEM((chunks, CHUNK), jnp.int32),
      pltpu.VMEM((NSLOT * CHUNK, 128), jnp.float32),
      pltpu.VMEM_SHARED((n_pad, 128), jnp.float32),
  ] + [pltpu.SemaphoreType.DMA] * (2 * NSLOT)
  if with_deg:
    out_type.append(jax.ShapeDtypeStruct((NW, n_pad), jnp.float32))
    scratch.append(pltpu.VMEM((n_pad,), jnp.float32))

  def body(*refs):
    if with_deg:
      (feat, srcp, dstp, p_out, deg_out,
       src_v, dst_v, buf0, acc_sh, *rest) = refs
    else:
      (feat, srcp, dstp, p_out,
       src_v, dst_v, buf0, acc_sh, *rest) = refs
    gs = rest[:NSLOT]
    ss = rest[NSLOT:2 * NSLOT]
    if with_deg:
      deg_v = rest[2 * NSLOT]
    slots = [buf0.at[pl.ds(b * CHUNK, CHUNK)] for b in range(NSLOT)]

    c = lax.axis_index("c")
    s = lax.axis_index("s")
    wid = c * NS + s
    r0 = s * rps

    pltpu.sync_copy(srcp.at[wid], src_v)
    pltpu.sync_copy(dstp.at[wid], dst_v)

    z16 = jnp.zeros((16,), jnp.float32)

    bufrows = NSLOT * CHUNK

    def zrow(i, carry):
      for k in range(8):
        buf0[i, pl.ds(k * 16, 16)] = z16
      return carry

    lax.fori_loop(0, bufrows, zrow, 0)
    for i in range(rps // bufrows):
      pltpu.sync_copy(buf0, acc_sh.at[pl.ds(r0 + i * bufrows, bufrows)])

    if with_deg:
      def dzrow(i, carry):
        deg_v[pl.ds(i * 16, 16)] = z16
        return carry

      lax.fori_loop(0, n_pad // 16, dzrow, 0)

    plsc.subcore_barrier()

    one16 = jnp.ones((16,), jnp.float32)

    def gstart(j, b):
      pltpu.async_copy(feat.at[src_v.at[j]], slots[b], gs[b])

    def gwait(j, b):
      pltpu.make_async_copy(feat.at[src_v.at[j]], slots[b], gs[b]).wait()

    def sstart(j, b):
      pltpu.async_copy(slots[b], acc_sh.at[dst_v.at[j]], ss[b], add=True)

    def swait(j, b):
      pltpu.make_async_copy(slots[b], acc_sh.at[dst_v.at[j]], ss[b]).wait()

    def deg_hist(j):
      if with_deg:
        for k in range(CHUNK // 16):
          idx = dst_v[j, pl.ds(k * 16, 16)]
          plsc.addupdate_scatter(deg_v, [idx], one16)

    gstart(0, 0)
    gstart(1, 1)

    def step(i, carry):
      j0 = NSLOT * i
      for b in range(NSLOT):
        j = j0 + b
        gwait(j, b)
        sstart(j, b)
        deg_hist(j)
        swait(j, b)

        @pl.when(j + 2 < chunks)
        def _():
          gstart(j + 2, b)

      return carry

    lax.fori_loop(0, chunks // NSLOT, step, 0)

    plsc.subcore_barrier()

    pltpu.sync_copy(acc_sh.at[pl.ds(r0, rps)], p_out.at[c, pl.ds(r0, rps)])
    if with_deg:
      pltpu.sync_copy(deg_v, deg_out.at[wid])

  return pl.kernel(body, out_type=tuple(out_type), mesh=mesh,
                   scratch_types=tuple(scratch),
                   compiler_params=pltpu.CompilerParams(
                       use_tc_tiling_on_sc=False,
                       needs_layout_passes=False))


def _mm_body(p_ref, dg_ref, w1_ref, b1_ref, w2_ref, o_ref):
  deg = jnp.maximum(jnp.sum(dg_ref[...], axis=0), 1.0)
  inv = (1.0 / deg)[:, None]
  agg = (p_ref[0] + p_ref[1]) * inv
  h = jnp.dot(agg, w1_ref[...], preferred_element_type=jnp.float32)
  h = jnp.maximum(h + b1_ref[...], 0.0)
  o_ref[...] = jnp.dot(h, w2_ref[...], preferred_element_type=jnp.float32)


def _fin_body(q_ref, dg_ref, b2_ref, o_ref):
  deg = jnp.maximum(jnp.sum(dg_ref[...], axis=0), 1.0)
  inv = (1.0 / deg)[:, None]
  o_ref[...] = (q_ref[0] + q_ref[1]) * inv + b2_ref[...]


def kernel(x, edge_index, W1, b1, W2, b2):
  n = x.shape[0]
  e = edge_index.shape[1]
  n_pad = pl.cdiv(n, NS * CHUNK) * NS * CHUNK
  if n_pad == n:
    n_pad += NS * CHUNK
  chunks = pl.cdiv(pl.cdiv(e, NW * CHUNK), NSLOT) * NSLOT
  e_pad = NW * CHUNK * chunks

  src = edge_index[0]
  dst = edge_index[1]
  pidx = jnp.arange(e_pad - e, dtype=jnp.int32)
  pad_src = pidx % n
  pad_dst = n + pidx % (n_pad - n)
  srcp = jnp.concatenate([src, pad_src]).reshape(NW, chunks, CHUNK)
  dstp = jnp.concatenate([dst, pad_dst]).reshape(NW, chunks, CHUNK)

  agg_deg = _make_agg(chunks, n_pad, with_deg=True)
  agg = _make_agg(chunks, n_pad, with_deg=False)

  p, degw = agg_deg(x, srcp, dstp)

  rb = 1280
  grid = (n_pad // rb,)
  h2 = pl.pallas_call(
      _mm_body,
      grid=grid,
      in_specs=[
          pl.BlockSpec((NC, rb, 128), lambda i: (0, i, 0)),
          pl.BlockSpec((NW, rb), lambda i: (0, i)),
          pl.BlockSpec((128, 128), lambda i: (0, 0)),
          pl.BlockSpec((1, 128), lambda i: (0, 0)),
          pl.BlockSpec((128, 128), lambda i: (0, 0)),
      ],
      out_specs=pl.BlockSpec((rb, 128), lambda i: (i, 0)),
      out_shape=jax.ShapeDtypeStruct((n_pad, 128), jnp.float32),
  )(p, degw, W1, b1.reshape(1, 128), W2)

  (q,) = agg(h2, srcp, dstp)

  out = pl.pallas_call(
      _fin_body,
      grid=grid,
      in_specs=[
          pl.BlockSpec((NC, rb, 128), lambda i: (0, i, 0)),
          pl.BlockSpec((NW, rb), lambda i: (0, i)),
          pl.BlockSpec((1, 128), lambda i: (0, 0)),
      ],
      out_specs=pl.BlockSpec((rb, 128), lambda i: (i, 0)),
      out_shape=jax.ShapeDtypeStruct((n_pad, 128), jnp.float32),
  )(q, degw, b2.reshape(1, 128))

  return out[:n]


# speedup vs baseline: 1.2247x; 1.0541x over previous
import jax
import jax.numpy as jnp
from jax import lax
from jax.experimental import pallas as pl
from jax.experimental.pallas import tpu as pltpu
from jax.experimental.pallas import tpu_sc as plsc

NC = 2
NS = 16
NW = NC * NS
CHUNK = 48
NSLOT = 3


def _make_agg(chunks, n_pad, with_deg):
  rps = n_pad // NS
  assert rps % 128 == 0
  mesh = plsc.VectorSubcoreMesh(core_axis_name="c", subcore_axis_name="s")

  out_type = [jax.ShapeDtypeStruct((NC, n_pad, 128), jnp.float32)]
  scratch = [
      pltpu.VMEM((chunks, CHUNK), jnp.int32),
      pltpu.VMEM((chunks, CHUNK), jnp.int32),
      pltpu.VMEM((NSLOT * CHUNK, 128), jnp.float32),
      pltpu.VMEM_SHARED((n_pad, 128), jnp.float32),
  ] + [pltpu.SemaphoreType.DMA] * (2 * NSLOT)
  if with_deg:
    out_type.append(jax.ShapeDtypeStruct((NW, n_pad), jnp.float32))
    scratch.append(pltpu.VMEM((n_pad,), jnp.float32))

  def body(*refs):
    if with_deg:
      (feat, srcp, dstp, p_out, deg_out,
       src_v, dst_v, buf0, acc_sh, *rest) = refs
    else:
      (feat, srcp, dstp, p_out,
       src_v, dst_v, buf0, acc_sh, *rest) = refs
    gs = rest[:NSLOT]
    ss = rest[NSLOT:2 * NSLOT]
    if with_deg:
      deg_v = rest[2 * NSLOT]
    slots = [buf0.at[pl.ds(b * CHUNK, CHUNK)] for b in range(NSLOT)]

    c = lax.axis_index("c")
    s = lax.axis_index("s")
    wid = c * NS + s
    r0 = s * rps

    pltpu.sync_copy(srcp.at[wid], src_v)
    pltpu.sync_copy(dstp.at[wid], dst_v)

    z16 = jnp.zeros((16,), jnp.float32)

    bufrows = NSLOT * CHUNK

    def zrow(i, carry):
      for k in range(8):
        buf0[i, pl.ds(k * 16, 16)] = z16
      return carry

    lax.fori_loop(0, bufrows, zrow, 0)
    zrows = min(bufrows, 128)
    for i in range(rps // zrows):
      pltpu.sync_copy(buf0.at[pl.ds(0, zrows)],
                      acc_sh.at[pl.ds(r0 + i * zrows, zrows)])

    if with_deg:
      def dzrow(i, carry):
        deg_v[pl.ds(i * 16, 16)] = z16
        return carry

      lax.fori_loop(0, n_pad // 16, dzrow, 0)

    plsc.subcore_barrier()

    one16 = jnp.ones((16,), jnp.float32)

    def gstart(j, b):
      pltpu.async_copy(feat.at[src_v.at[j]], slots[b], gs[b])

    def gwait(j, b):
      pltpu.make_async_copy(feat.at[src_v.at[j]], slots[b], gs[b]).wait()

    def sstart(j, b):
      pltpu.async_copy(slots[b], acc_sh.at[dst_v.at[j]], ss[b], add=True)

    def swait(j, b):
      pltpu.make_async_copy(slots[b], acc_sh.at[dst_v.at[j]], ss[b]).wait()

    def deg_hist(j):
      if with_deg:
        for k in range(CHUNK // 16):
          idx = dst_v[j, pl.ds(k * 16, 16)]
          plsc.addupdate_scatter(deg_v, [idx], one16)

    gstart(0, 0)
    gstart(1, 1)

    def step(i, carry):
      j0 = NSLOT * i
      for b in range(NSLOT):
        j = j0 + b
        gwait(j, b)
        sstart(j, b)
        deg_hist(j)
        b2 = (b + 2) % NSLOT
        jprev = j + 2 - NSLOT

        @pl.when(jprev >= 0)
        def _():
          swait(jprev, b2)

        @pl.when(j + 2 < chunks)
        def _():
          gstart(j + 2, b2)

      return carry

    lax.fori_loop(0, chunks // NSLOT, step, 0)
    for jt in range(chunks - NSLOT + 2, chunks):
      swait(jt, jt % NSLOT)

    plsc.subcore_barrier()

    pltpu.sync_copy(acc_sh.at[pl.ds(r0, rps)], p_out.at[c, pl.ds(r0, rps)])
    if with_deg:
      pltpu.sync_copy(deg_v, deg_out.at[wid])

  return pl.kernel(body, out_type=tuple(out_type), mesh=mesh,
                   scratch_types=tuple(scratch),
                   compiler_params=pltpu.CompilerParams(
                       use_tc_tiling_on_sc=False,
                       needs_layout_passes=False))


def _mm_body(p_ref, dg_ref, w1_ref, b1_ref, w2_ref, o_ref):
  deg = jnp.maximum(jnp.sum(dg_ref[...], axis=0), 1.0)
  inv = (1.0 / deg)[:, None]
  agg = (p_ref[0] + p_ref[1]) * inv
  h = jnp.dot(agg, w1_ref[...], preferred_element_type=jnp.float32)
  h = jnp.maximum(h + b1_ref[...], 0.0)
  o_ref[...] = jnp.dot(h, w2_ref[...], preferred_element_type=jnp.float32)


def _fin_body(q_ref, dg_ref, b2_ref, o_ref):
  deg = jnp.maximum(jnp.sum(dg_ref[...], axis=0), 1.0)
  inv = (1.0 / deg)[:, None]
  o_ref[...] = (q_ref[0] + q_ref[1]) * inv + b2_ref[...]


def kernel(x, edge_index, W1, b1, W2, b2):
  n = x.shape[0]
  e = edge_index.shape[1]
  n_pad = pl.cdiv(n, NS * 128) * NS * 128
  if n_pad == n:
    n_pad += NS * 128
  chunks = pl.cdiv(pl.cdiv(e, NW * CHUNK), NSLOT) * NSLOT
  e_pad = NW * CHUNK * chunks

  src = edge_index[0]
  dst = edge_index[1]
  pidx = jnp.arange(e_pad - e, dtype=jnp.int32)
  pad_src = pidx % n
  pad_dst = n + pidx % (n_pad - n)
  srcp = jnp.concatenate([src, pad_src]).reshape(NW, chunks, CHUNK)
  dstp = jnp.concatenate([dst, pad_dst]).reshape(NW, chunks, CHUNK)

  agg_deg = _make_agg(chunks, n_pad, with_deg=True)
  agg = _make_agg(chunks, n_pad, with_deg=False)

  p, degw = agg_deg(x, srcp, dstp)

  rb = 1280
  grid = (n_pad // rb,)
  h2 = pl.pallas_call(
      _mm_body,
      grid=grid,
      in_specs=[
          pl.BlockSpec((NC, rb, 128), lambda i: (0, i, 0)),
          pl.BlockSpec((NW, rb), lambda i: (0, i)),
          pl.BlockSpec((128, 128), lambda i: (0, 0)),
          pl.BlockSpec((1, 128), lambda i: (0, 0)),
          pl.BlockSpec((128, 128), lambda i: (0, 0)),
      ],
      out_specs=pl.BlockSpec((rb, 128), lambda i: (i, 0)),
      out_shape=jax.ShapeDtypeStruct((n_pad, 128), jnp.float32),
  )(p, degw, W1, b1.reshape(1, 128), W2)

  (q,) = agg(h2, srcp, dstp)

  out = pl.pallas_call(
      _fin_body,
      grid=grid,
      in_specs=[
          pl.BlockSpec((NC, rb, 128), lambda i: (0, i, 0)),
          pl.BlockSpec((NW, rb), lambda i: (0, i)),
          pl.BlockSpec((1, 128), lambda i: (0, 0)),
      ],
      out_specs=pl.BlockSpec((rb, 128), lambda i: (i, 0)),
      out_shape=jax.ShapeDtypeStruct((n_pad, 128), jnp.float32),
  )(q, degw, b2.reshape(1, 128))

  return out[:n]
